# Initial kernel scaffold; baseline (speedup 1.0000x reference)
import jax
import jax.numpy as jnp
from jax.experimental import pallas as pl


def kernel(x, edge_index0, edge_index1, edge_index2, D0_row, D0_col, D0_val, D1_row, D1_col, D1_val, W0, b0, W1, b1, W2, b2, Wlin, blin):
    raise NotImplementedError("write your pallas kernel here")


# speedup vs baseline: 7.1381x; 7.1381x over previous
import functools

import jax
import jax.numpy as jnp
from jax import lax
from jax.experimental import pallas as pl
from jax.experimental.pallas import tpu as pltpu
from jax.experimental.pallas import tpu_sc as plsc

_f32 = jnp.float32
_i32 = jnp.int32

_KCH = 6
_NT = 16

_NP0, _NP1, _NP2 = 10240, 2560, 1024
_F0, _F1, _F2 = 16, 64, 128

def _rsqrt16(x):
    i = plsc.bitcast(x, _i32)
    i = 0x5F3759DF - jnp.right_shift(i, 1)
    y = plsc.bitcast(i, _f32)
    for _ in range(3):
        y = y * (1.5 - 0.5 * x * y * y)
    return jnp.where(x > 0.0, y, 0.0)


def _zchunks(rows):
    out, off = [], 0
    while off < rows:
        sz = min(128, rows - off)
        out.append((off, sz))
        off += sz
    return out


def _build_cheb_kernel(np_, f, nec, pool_cfg, iota_shape):
    rpt = np_ // _NT
    nr = np_ // 16
    drows = nr // _NT
    zlist = _zchunks(rpt)
    nzc, czs = iota_shape
    nv = f // 16
    nec2 = nec // 2

    mesh = plsc.VectorSubcoreMesh(core_axis_name="c", subcore_axis_name="s",
                                  num_cores=1)

    scratch = [
        pltpu.VMEM((nec, 128), _i32),
        pltpu.VMEM((nec, 128), _i32),
        pltpu.VMEM((nzc, czs), _i32),
        pltpu.VMEM((128, f), _f32),
        pltpu.VMEM((128, f), _f32),
        pltpu.VMEM((128, f), _f32),
        pltpu.VMEM((rpt, f), _f32),
        pltpu.VMEM((rpt, f), _f32),
        pltpu.VMEM((rpt, f), _f32),
        pltpu.VMEM((rpt, 16), _f32),
        pltpu.VMEM((nr, 16), _f32),
        pltpu.VMEM_SHARED((np_, f), _f32),
        pltpu.VMEM_SHARED((nr, 16), _f32),
        pltpu.SemaphoreType.DMA,
        pltpu.SemaphoreType.DMA,
    ]
    if pool_cfg is not None:
        np_prev, n_pc, cp = pool_cfg
        rptp = np_prev // _NT
        scratch.append(pltpu.VMEM((n_pc, cp), _i32))

    out_type = (jax.ShapeDtypeStruct((_KCH, np_, f), _f32),
                jax.ShapeDtypeStruct((np_, f), _f32))

    def body(*refs):
        if pool_cfg is None:
            (hin, srcp, dstp, iotah, txs, sbuf,
             src_v, dst_v, iota_v, rb0, rb1, zbuf, txA, txB, abuf,
             dinvb, degp, acc, degsh, s0, s1) = refs
            prow_v = None
        else:
            (hin, poolrow, srcp, dstp, iotah, txs, sbuf,
             src_v, dst_v, iota_v, rb0, rb1, zbuf, txA, txB, abuf,
             dinvb, degp, acc, degsh, s0, s1, prow_v) = refs

        wid = lax.axis_index("s")
        z16f = jnp.zeros((16,), _f32)
        ones16 = jnp.full((16,), 1.0, _f32)

        pltpu.sync_copy(srcp.at[wid], src_v)
        pltpu.sync_copy(dstp.at[wid], dst_v)
        pltpu.sync_copy(iotah, iota_v)
        if pool_cfg is not None:
            pltpu.sync_copy(poolrow.at[wid], prow_v)

        @pl.loop(0, 128)
        def _(r):
            for v in range(nv):
                zbuf[r, pl.ds(16 * v, 16)] = z16f

        @pl.loop(0, nr)
        def _(r):
            degp[r] = z16f

        @pl.loop(0, nec)
        def _(j):
            for g in range(8):
                d = dst_v[j, pl.ds(16 * g, 16)]
                row = jnp.right_shift(d, 4)
                lane = jnp.bitwise_and(d, 15)
                plsc.addupdate_scatter(degp, [row, lane], ones16)

        pltpu.sync_copy(zbuf.at[pl.ds(0, drows), pl.ds(0, 16)],
                        degsh.at[pl.ds(wid * drows, drows)])
        plsc.subcore_barrier()
        for z in range(nzc):
            pltpu.sync_copy(degp.at[pl.ds(z * czs, czs)],
                            degsh.at[iota_v.at[z]], add=True)
        plsc.subcore_barrier()

        pltpu.sync_copy(degsh.at[pl.ds(wid * drows, drows)],
                        degp.at[pl.ds(0, drows)])

        @pl.loop(0, drows)
        def _(r):
            degp[r] = _rsqrt16(degp[r])

        @pl.loop(0, rpt)
        def _(r):
            ir = jnp.full((16,), jnp.right_shift(r, 4), _i32)
            il = jnp.full((16,), jnp.bitwise_and(r, 15), _i32)
            dinvb[r] = plsc.load_gather(degp, [ir, il])

        if pool_cfg is None:
            pltpu.sync_copy(hin.at[pl.ds(wid * rpt, rpt)], txA)
        else:
            for off, sz in zlist:
                pltpu.sync_copy(zbuf.at[pl.ds(0, sz)],
                                acc.at[pl.ds(wid * rpt + off, sz)])
            plsc.subcore_barrier()
            for pc in range(n_pc):
                pltpu.sync_copy(hin.at[pl.ds(wid * rptp + pc * cp, cp)],
                                rb0.at[pl.ds(0, cp)])
                pltpu.sync_copy(rb0.at[pl.ds(0, cp)],
                                acc.at[prow_v.at[pc]], add=True)
            plsc.subcore_barrier()
            pltpu.sync_copy(acc.at[pl.ds(wid * rpt, rpt)], txA)

        pltpu.sync_copy(txA, txs.at[0, pl.ds(wid * rpt, rpt)])

        @pl.loop(0, rpt)
        def _(r):
            d = dinvb[r]
            for v in range(nv):
                sl = pl.ds(16 * v, 16)
                abuf[r, sl] = d * txA[r, sl]

        pltpu.sync_copy(abuf, sbuf.at[pl.ds(wid * rpt, rpt)])
        plsc.subcore_barrier()

        bufs = [txA, txB]
        for k in range(1, _KCH):
            for off, sz in zlist:
                pltpu.sync_copy(zbuf.at[pl.ds(0, sz)],
                                acc.at[pl.ds(wid * rpt + off, sz)])
            plsc.subcore_barrier()

            pltpu.async_copy(sbuf.at[src_v.at[0]], rb0, s0)

            @pl.loop(0, nec2)
            def _(p):
                j = p * 2
                pltpu.async_copy(sbuf.at[src_v.at[j + 1]], rb1, s1)
                pltpu.make_async_copy(sbuf.at[src_v.at[j]], rb0, s0).wait()
                pltpu.sync_copy(rb0, acc.at[dst_v.at[j]], add=True)

                @pl.when(p + 1 < nec2)
                def _():
                    pltpu.async_copy(sbuf.at[src_v.at[j + 2]], rb0, s0)

                pltpu.make_async_copy(sbuf.at[src_v.at[j + 1]], rb1, s1).wait()
                pltpu.sync_copy(rb1, acc.at[dst_v.at[j + 1]], add=True)

            plsc.subcore_barrier()

            pltpu.sync_copy(acc.at[pl.ds(wid * rpt, rpt)], abuf)
            told, tnew = bufs[(k + 1) % 2], bufs[k % 2]
            last = k == _KCH - 1

            @pl.loop(0, rpt)
            def _(r):
                d = dinvb[r]
                for v in range(nv):
                    sl = pl.ds(16 * v, 16)
                    a = abuf[r, sl]
                    if k == 1:
                        tn = -(d * a)
                    else:
                        tn = (-2.0) * d * a - tnew[r, sl]
                    tnew[r, sl] = tn
                    if not last:
                        abuf[r, sl] = d * tn

            pltpu.sync_copy(tnew, txs.at[k, pl.ds(wid * rpt, rpt)])
            if not last:
                pltpu.sync_copy(abuf, sbuf.at[pl.ds(wid * rpt, rpt)])
                plsc.subcore_barrier()

    params = pltpu.CompilerParams(needs_layout_passes=False,
                                  use_tc_tiling_on_sc=False)
    return functools.partial(pl.kernel, out_type=out_type, mesh=mesh,
                             scratch_types=scratch,
                             compiler_params=params)(body)


def _build_dense_kernel(np_, f, fo, bn):
    grid = np_ // bn

    def body(txs_ref, w_ref, b_ref, s_ref, out_ref):
        acc = jnp.zeros((bn, fo), _f32)
        for k in range(_KCH):
            acc += jnp.dot(txs_ref[k], w_ref[k],
                           preferred_element_type=_f32)
        acc = jnp.maximum(acc + b_ref[...], 0.0)
        out_ref[...] = acc * s_ref[...]

    return pl.pallas_call(
        body,
        grid=(grid,),
        in_specs=[
            pl.BlockSpec((_KCH, bn, f), lambda i: (0, i, 0)),
            pl.BlockSpec((_KCH, f, fo), lambda i: (0, 0, 0)),
            pl.BlockSpec((1, fo), lambda i: (0, 0)),
            pl.BlockSpec((bn, 1), lambda i: (i, 0)),
        ],
        out_specs=pl.BlockSpec((bn, fo), lambda i: (i, 0)),
        out_shape=jax.ShapeDtypeStruct((np_, fo), _f32),
    )


def _build_final_kernel(np_, f, fo, bn, ncls):
    grid = np_ // bn

    def body(txs_ref, w_ref, b_ref, wl_ref, oh_ref, bl_ref, out_ref):
        i = pl.program_id(0)
        h = jnp.zeros((bn, fo), _f32)
        for k in range(_KCH):
            h += jnp.dot(txs_ref[k], w_ref[k], preferred_element_type=_f32)
        h = h + b_ref[...]
        z = jnp.zeros((1, 128), _f32)
        for o in range(ncls):
            s = jnp.sum(h * wl_ref[o])
            z += s * oh_ref[o:o + 1, :]

        @pl.when(i == 0)
        def _():
            out_ref[...] = jnp.broadcast_to(bl_ref[...], (8, 128))

        out_ref[...] += jnp.broadcast_to(z, (8, 128))

    return pl.pallas_call(
        body,
        grid=(grid,),
        in_specs=[
            pl.BlockSpec((_KCH, bn, f), lambda i: (0, i, 0)),
            pl.BlockSpec((_KCH, f, fo), lambda i: (0, 0, 0)),
            pl.BlockSpec((1, fo), lambda i: (0, 0)),
            pl.BlockSpec((ncls, bn, fo), lambda i: (0, i, 0)),
            pl.BlockSpec((ncls, 128), lambda i: (0, 0)),
            pl.BlockSpec((1, 128), lambda i: (0, 0)),
        ],
        out_specs=pl.BlockSpec((8, 128), lambda i: (0, 0)),
        out_shape=jax.ShapeDtypeStruct((8, 128), _f32),
    )


def _pad_edges(ei, nec_total, trash):
    e = ei.shape[1]
    ep = _NT * nec_total * 128
    pad = jnp.full((2, ep - e), trash, _i32)
    full = jnp.concatenate([ei, pad], axis=1)
    return (full[0].reshape(_NT, nec_total, 128),
            full[1].reshape(_NT, nec_total, 128))


def _pad_rows(idx, np_prev, n_pc, cp, trash):
    p = jnp.full((np_prev - idx.shape[0],), trash, _i32)
    return jnp.concatenate([idx, p]).reshape(_NT, n_pc, cp)


def _pad_val(v, np_prev):
    return jnp.concatenate([v, jnp.zeros((np_prev - v.shape[0],), _f32)])


_cheb0 = _build_cheb_kernel(_NP0, _F0, 80, None, (5, 128))
_cheb1 = _build_cheb_kernel(_NP1, _F1, 20, (_NP0, 5, 128), (5, 32))
_cheb2 = _build_cheb_kernel(_NP2, _F2, 6, (_NP1, 5, 32), (2, 32))
_dense0 = _build_dense_kernel(_NP0, _F0, 64, 1024)
_dense1 = _build_dense_kernel(_NP1, _F1, 128, 512)
_final = _build_final_kernel(_NP2, _F2, 256, 128, 10)

def kernel(x, edge_index0, edge_index1, edge_index2, D0_row, D0_col, D0_val,
           D1_row, D1_col, D1_val, W0, b0, W1, b1, W2, b2, Wlin, blin):
    n0, n1, n2 = 10000, 2500, 625
    iota0 = jnp.arange(_NP0 // 16, dtype=_i32).reshape(5, 128)
    iota1 = jnp.arange(_NP1 // 16, dtype=_i32).reshape(5, 32)
    iota2 = jnp.arange(_NP2 // 16, dtype=_i32).reshape(2, 32)

    xp = jnp.zeros((_NP0, _F0), _f32).at[:n0, :3].set(x)
    src0, dst0 = _pad_edges(edge_index0, 80, n0)
    src1, dst1 = _pad_edges(edge_index1, 20, n1)
    src2, dst2 = _pad_edges(edge_index2, 6, n2)

    txs0, _ = _cheb0(xp, src0, dst0, iota0)

    w0p = jnp.zeros((_KCH, _F0, 64), _f32).at[:, :3, :].set(W0)
    h0 = _dense0(txs0, w0p, b0.reshape(1, 64),
                 _pad_val(D0_val, _NP0).reshape(_NP0, 1))

    pr0 = _pad_rows(D0_row, _NP0, 5, 128, n1)
    txs1, _ = _cheb1(h0, pr0, src1, dst1, iota1)

    h1 = _dense1(txs1, W1, b1.reshape(1, 128),
                 _pad_val(D1_val, _NP1).reshape(_NP1, 1))

    pr1 = _pad_rows(D1_row, _NP1, 5, 32, n2)
    txs2, _ = _cheb2(h1, pr1, src2, dst2, iota2)

    wlt = jnp.transpose(Wlin.reshape(n2, 256, 10), (2, 0, 1))
    wlt = jnp.zeros((10, _NP2, 256), _f32).at[:, :n2, :].set(wlt)
    oh = jnp.eye(10, 128, dtype=_f32)
    bl = jnp.zeros((1, 128), _f32).at[0, :10].set(blin)

    z8 = _final(txs2, W2, b2.reshape(1, 256), wlt, oh, bl)
    return z8[0, :10]


# speedup vs baseline: 10.8958x; 1.5264x over previous
import functools

import jax
import jax.numpy as jnp
from jax import lax
from jax.experimental import pallas as pl
from jax.experimental.pallas import tpu as pltpu
from jax.experimental.pallas import tpu_sc as plsc

_f32 = jnp.float32
_i32 = jnp.int32

_KCH = 6
_NT = 16

_NP0, _NP1, _NP2 = 10240, 2560, 1024
_F0, _F1, _F2 = 16, 64, 128


def _rsqrt16(x):
    i = plsc.bitcast(x, _i32)
    i = 0x5F3759DF - jnp.right_shift(i, 1)
    y = plsc.bitcast(i, _f32)
    for _ in range(3):
        y = y * (1.5 - 0.5 * x * y * y)
    return jnp.where(x > 0.0, y, 0.0)


def _zchunks(rows):
    out, off = [], 0
    while off < rows:
        sz = min(128, rows - off)
        out.append((off, sz))
        off += sz
    return out


def _build_cheb_kernel(np_, f, nec, pool_cfg, iota_shape, split):
    ncores = 2 if split else 1
    fh = f // ncores
    rpt = np_ // _NT
    nr = np_ // 16
    drows = nr // _NT
    zlist = _zchunks(rpt)
    nzc, czs = iota_shape
    nv = fh // 16
    nb = {80: 8, 20: 4, 6: 3}[nec]
    nq = nec // nb

    mesh = plsc.VectorSubcoreMesh(core_axis_name="c", subcore_axis_name="s",
                                  num_cores=ncores)

    scratch = [
        pltpu.VMEM((nec, 128), _i32),
        pltpu.VMEM((nec, 128), _i32),
        pltpu.VMEM((nzc, czs), _i32),
    ] + [pltpu.VMEM((128, fh), _f32) for _ in range(nb)] + [
        pltpu.VMEM((128, fh), _f32),
        pltpu.VMEM((rpt, fh), _f32),
        pltpu.VMEM((rpt, fh), _f32),
        pltpu.VMEM((rpt, fh), _f32),
        pltpu.VMEM((rpt, 16), _f32),
        pltpu.VMEM((nr, 16), _f32),
        pltpu.VMEM_SHARED((np_, fh), _f32),
        pltpu.VMEM_SHARED((nr, 16), _f32),
    ] + [pltpu.SemaphoreType.DMA for _ in range(2 * nb)]
    if pool_cfg is not None:
        np_prev, n_pc, cp = pool_cfg
        rptp = np_prev // _NT
        scratch.append(pltpu.VMEM((n_pc, cp), _i32))

    if split:
        out_type = (jax.ShapeDtypeStruct((2, _KCH, np_, fh), _f32),
                    jax.ShapeDtypeStruct((2, np_, fh), _f32))
    else:
        out_type = (jax.ShapeDtypeStruct((_KCH, np_, fh), _f32),
                    jax.ShapeDtypeStruct((np_, fh), _f32))

    def body(*refs):
        if pool_cfg is None:
            hin, srcp, dstp, iotah, txs, sbuf = refs[:6]
            rest = refs[6:]
            prow_v = None
        else:
            hin, poolrow, srcp, dstp, iotah, txs, sbuf = refs[:7]
            rest = refs[7:]
        src_v, dst_v, iota_v = rest[:3]
        rbs = rest[3:3 + nb]
        zbuf, txA, txB, abuf, dinvb, degp, acc, degsh = rest[3 + nb:11 + nb]
        gsems = rest[11 + nb:11 + 2 * nb]
        ssems = rest[11 + 2 * nb:11 + 3 * nb]
        if pool_cfg is not None:
            prow_v = rest[11 + 3 * nb]
        rb0 = rbs[0]

        wid = lax.axis_index("s")
        if split:
            cid = lax.axis_index("c")
            stab = sbuf.at[cid]

            def tx_dst(k):
                return txs.at[cid, k, pl.ds(wid * rpt, rpt)]

            s_dst = sbuf.at[cid, pl.ds(wid * rpt, rpt)]
        else:
            stab = sbuf

            def tx_dst(k):
                return txs.at[k, pl.ds(wid * rpt, rpt)]

            s_dst = sbuf.at[pl.ds(wid * rpt, rpt)]
        z16f = jnp.zeros((16,), _f32)
        ones16 = jnp.full((16,), 1.0, _f32)

        pltpu.sync_copy(srcp.at[wid], src_v)
        pltpu.sync_copy(dstp.at[wid], dst_v)
        pltpu.sync_copy(iotah, iota_v)
        if pool_cfg is not None:
            pltpu.sync_copy(poolrow.at[wid], prow_v)

        @pl.loop(0, 128, unroll=8)
        def _(r):
            for v in range(nv):
                zbuf[r, pl.ds(16 * v, 16)] = z16f

        @pl.loop(0, nr, unroll=8)
        def _(r):
            degp[r] = z16f

        @pl.loop(0, nec)
        def _(j):
            for g in range(8):
                d = dst_v[j, pl.ds(16 * g, 16)]
                row = jnp.right_shift(d, 4)
                lane = jnp.bitwise_and(d, 15)
                plsc.addupdate_scatter(degp, [row, lane], ones16)

        pltpu.sync_copy(zbuf.at[pl.ds(0, drows), pl.ds(0, 16)],
                        degsh.at[pl.ds(wid * drows, drows)])
        plsc.subcore_barrier()
        for z in range(nzc):
            pltpu.sync_copy(degp.at[pl.ds(z * czs, czs)],
                            degsh.at[iota_v.at[z]], add=True)
        plsc.subcore_barrier()

        pltpu.sync_copy(degsh.at[pl.ds(wid * drows, drows)],
                        degp.at[pl.ds(0, drows)])

        @pl.loop(0, drows, unroll=4)
        def _(r):
            degp[r] = _rsqrt16(degp[r])

        @pl.loop(0, rpt, unroll=8)
        def _(r):
            ir = jnp.full((16,), jnp.right_shift(r, 4), _i32)
            il = jnp.full((16,), jnp.bitwise_and(r, 15), _i32)
            dinvb[r] = plsc.load_gather(degp, [ir, il])

        if pool_cfg is None:
            pltpu.sync_copy(hin.at[pl.ds(wid * rpt, rpt)], txA)
        else:
            for off, sz in zlist:
                pltpu.sync_copy(zbuf.at[pl.ds(0, sz)],
                                acc.at[pl.ds(wid * rpt + off, sz)])
            plsc.subcore_barrier()
            for pc in range(n_pc):
                hsrc = (hin.at[cid, pl.ds(wid * rptp + pc * cp, cp)] if split
                        else hin.at[pl.ds(wid * rptp + pc * cp, cp)])
                pltpu.sync_copy(hsrc, rb0.at[pl.ds(0, cp)])
                pltpu.sync_copy(rb0.at[pl.ds(0, cp)],
                                acc.at[prow_v.at[pc]], add=True)
            plsc.subcore_barrier()
            pltpu.sync_copy(acc.at[pl.ds(wid * rpt, rpt)], txA)

        pltpu.sync_copy(txA, tx_dst(0))

        @pl.loop(0, rpt, unroll=8)
        def _(r):
            d = dinvb[r]
            for v in range(nv):
                sl = pl.ds(16 * v, 16)
                abuf[r, sl] = d * txA[r, sl]

        pltpu.sync_copy(abuf, s_dst)
        plsc.subcore_barrier()

        bufs = [txA, txB]
        for k in range(1, _KCH):
            for off, sz in zlist:
                pltpu.sync_copy(zbuf.at[pl.ds(0, sz)],
                                acc.at[pl.ds(wid * rpt + off, sz)])
            plsc.subcore_barrier()

            for b in range(nb):
                pltpu.async_copy(stab.at[src_v.at[b]], rbs[b], gsems[b])

            @pl.loop(0, nq)
            def _(q):
                base = q * nb
                for b in range(nb):
                    j = base + b
                    pltpu.make_async_copy(stab.at[src_v.at[j]], rbs[b],
                                          gsems[b]).wait()
                    pltpu.sync_copy(rbs[b], acc.at[dst_v.at[j]], add=True)

                    @pl.when(q + 1 < nq)
                    def _(b=b, j=j):
                        pltpu.async_copy(stab.at[src_v.at[j + nb]], rbs[b],
                                         gsems[b])

            plsc.subcore_barrier()

            pltpu.sync_copy(acc.at[pl.ds(wid * rpt, rpt)], abuf)
            tnew = bufs[k % 2]
            last = k == _KCH - 1

            @pl.loop(0, rpt, unroll=8)
            def _(r):
                d = dinvb[r]
                for v in range(nv):
                    sl = pl.ds(16 * v, 16)
                    a = abuf[r, sl]
                    if k == 1:
                        tn = -(d * a)
                    else:
                        tn = (-2.0) * d * a - tnew[r, sl]
                    tnew[r, sl] = tn
                    if not last:
                        abuf[r, sl] = d * tn

            pltpu.sync_copy(tnew, tx_dst(k))
            if not last:
                pltpu.sync_copy(abuf, s_dst)
                plsc.subcore_barrier()

    params = pltpu.CompilerParams(needs_layout_passes=False,
                                  use_tc_tiling_on_sc=False)
    return functools.partial(pl.kernel, out_type=out_type, mesh=mesh,
                             scratch_types=scratch,
                             compiler_params=params)(body)


def _build_dense_kernel(np_, f, fo, bn, split_in):
    grid = (2, np_ // bn)
    foh = fo // 2

    def body(txs_ref, w_ref, b_ref, s_ref, out_ref):
        acc = jnp.zeros((bn, foh), _f32)
        for k in range(_KCH):
            if split_in:
                acc += jnp.dot(txs_ref[0, k], w_ref[0, 0, k],
                               preferred_element_type=_f32)
                acc += jnp.dot(txs_ref[1, k], w_ref[0, 1, k],
                               preferred_element_type=_f32)
            else:
                acc += jnp.dot(txs_ref[k], w_ref[0, k],
                               preferred_element_type=_f32)
        acc = jnp.maximum(acc + b_ref[0], 0.0)
        out_ref[0] = acc * s_ref[...]

    if split_in:
        fin = f // 2
        tx_spec = pl.BlockSpec((2, _KCH, bn, fin), lambda j, i: (0, 0, i, 0))
        w_spec = pl.BlockSpec((1, 2, _KCH, fin, foh),
                              lambda j, i: (j, 0, 0, 0, 0))
    else:
        tx_spec = pl.BlockSpec((_KCH, bn, f), lambda j, i: (0, i, 0))
        w_spec = pl.BlockSpec((1, _KCH, f, foh), lambda j, i: (j, 0, 0, 0))

    return pl.pallas_call(
        body,
        grid=grid,
        in_specs=[
            tx_spec,
            w_spec,
            pl.BlockSpec((1, 1, foh), lambda j, i: (j, 0, 0)),
            pl.BlockSpec((bn, 1), lambda j, i: (i, 0)),
        ],
        out_specs=pl.BlockSpec((1, bn, foh), lambda j, i: (j, i, 0)),
        out_shape=jax.ShapeDtypeStruct((2, np_, foh), _f32),
    )


def _build_final_kernel(np_, f, fo, bn, ncls):
    grid = np_ // bn
    fin = f // 2

    def body(txs_ref, w_ref, b_ref, wl_ref, oh_ref, bl_ref, out_ref):
        i = pl.program_id(0)
        h = jnp.zeros((bn, fo), _f32)
        for k in range(_KCH):
            h += jnp.dot(txs_ref[0, k], w_ref[0, k],
                         preferred_element_type=_f32)
            h += jnp.dot(txs_ref[1, k], w_ref[1, k],
                         preferred_element_type=_f32)
        h = h + b_ref[...]
        z = jnp.zeros((1, 128), _f32)
        for o in range(ncls):
            s = jnp.sum(h * wl_ref[o])
            z += s * oh_ref[o:o + 1, :]

        @pl.when(i == 0)
        def _():
            out_ref[...] = jnp.broadcast_to(bl_ref[...], (8, 128))

        out_ref[...] += jnp.broadcast_to(z, (8, 128))

    return pl.pallas_call(
        body,
        grid=(grid,),
        in_specs=[
            pl.BlockSpec((2, _KCH, bn, fin), lambda i: (0, 0, i, 0)),
            pl.BlockSpec((2, _KCH, fin, fo), lambda i: (0, 0, 0, 0)),
            pl.BlockSpec((1, fo), lambda i: (0, 0)),
            pl.BlockSpec((ncls, bn, fo), lambda i: (0, i, 0)),
            pl.BlockSpec((ncls, 128), lambda i: (0, 0)),
            pl.BlockSpec((1, 128), lambda i: (0, 0)),
        ],
        out_specs=pl.BlockSpec((8, 128), lambda i: (0, 0)),
        out_shape=jax.ShapeDtypeStruct((8, 128), _f32),
    )


def _pad_edges(ei, nec_total, trash):
    e = ei.shape[1]
    ep = _NT * nec_total * 128
    pad = jnp.full((2, ep - e), trash, _i32)
    full = jnp.concatenate([ei, pad], axis=1)
    return (full[0].reshape(_NT, nec_total, 128),
            full[1].reshape(_NT, nec_total, 128))


def _pad_rows(idx, np_prev, n_pc, cp, trash):
    p = jnp.full((np_prev - idx.shape[0],), trash, _i32)
    return jnp.concatenate([idx, p]).reshape(_NT, n_pc, cp)


def _pad_val(v, np_prev):
    return jnp.concatenate([v, jnp.zeros((np_prev - v.shape[0],), _f32)])


_cheb0 = _build_cheb_kernel(_NP0, _F0, 80, None, (5, 128), False)
_cheb1 = _build_cheb_kernel(_NP1, _F1, 20, (_NP0, 5, 128), (5, 32), True)
_cheb2 = _build_cheb_kernel(_NP2, _F2, 6, (_NP1, 5, 32), (2, 32), True)
_dense0 = _build_dense_kernel(_NP0, _F0, 64, 1024, False)
_dense1 = _build_dense_kernel(_NP1, _F1, 128, 512, True)
_final = _build_final_kernel(_NP2, _F2, 256, 128, 10)


def kernel(x, edge_index0, edge_index1, edge_index2, D0_row, D0_col, D0_val,
           D1_row, D1_col, D1_val, W0, b0, W1, b1, W2, b2, Wlin, blin):
    n0, n1, n2 = 10000, 2500, 625
    iota0 = jnp.arange(_NP0 // 16, dtype=_i32).reshape(5, 128)
    iota1 = jnp.arange(_NP1 // 16, dtype=_i32).reshape(5, 32)
    iota2 = jnp.arange(_NP2 // 16, dtype=_i32).reshape(2, 32)

    xp = jnp.zeros((_NP0, _F0), _f32).at[:n0, :3].set(x)
    src0, dst0 = _pad_edges(edge_index0, 80, n0)
    src1, dst1 = _pad_edges(edge_index1, 20, n1)
    src2, dst2 = _pad_edges(edge_index2, 6, n2)

    txs0, _ = _cheb0(xp, src0, dst0, iota0)

    w0p = jnp.zeros((_KCH, _F0, 64), _f32).at[:, :3, :].set(W0)
    w0s = jnp.stack([w0p[:, :, :32], w0p[:, :, 32:]])
    b0s = jnp.stack([b0[:32].reshape(1, 32), b0[32:].reshape(1, 32)])
    h0 = _dense0(txs0, w0s, b0s, _pad_val(D0_val, _NP0).reshape(_NP0, 1))

    pr0 = _pad_rows(D0_row, _NP0, 5, 128, n1)
    txs1, _ = _cheb1(h0, pr0, src1, dst1, iota1)

    w1q = jnp.stack([
        jnp.stack([W1[:, :32, :64], W1[:, 32:, :64]]),
        jnp.stack([W1[:, :32, 64:], W1[:, 32:, 64:]]),
    ])
    b1s = jnp.stack([b1[:64].reshape(1, 64), b1[64:].reshape(1, 64)])
    h1 = _dense1(txs1, w1q, b1s, _pad_val(D1_val, _NP1).reshape(_NP1, 1))

    pr1 = _pad_rows(D1_row, _NP1, 5, 32, n2)
    txs2, _ = _cheb2(h1, pr1, src2, dst2, iota2)

    w2s = jnp.stack([W2[:, :64, :], W2[:, 64:, :]])
    wlt = jnp.transpose(Wlin.reshape(n2, 256, 10), (2, 0, 1))
    wlt = jnp.zeros((10, _NP2, 256), _f32).at[:, :n2, :].set(wlt)
    oh = jnp.eye(10, 128, dtype=_f32)
    bl = jnp.zeros((1, 128), _f32).at[0, :10].set(blin)

    z8 = _final(txs2, w2s, b2.reshape(1, 256), wlt, oh, bl)
    return z8[0, :10]


# speedup vs baseline: 20.9852x; 1.9260x over previous
import functools

import jax
import jax.numpy as jnp
from jax import lax
from jax.experimental import pallas as pl
from jax.experimental.pallas import tpu as pltpu
from jax.experimental.pallas import tpu_sc as plsc

_f32 = jnp.float32
_i32 = jnp.int32

_KCH = 6
_NT = 16

_NP0, _NP1, _NP2 = 10240, 2560, 1024
_F0, _F1, _F2 = 16, 64, 128


def _rsqrt16(x):
    i = plsc.bitcast(x, _i32)
    i = 0x5F3759DF - jnp.right_shift(i, 1)
    y = plsc.bitcast(i, _f32)
    for _ in range(3):
        y = y * (1.5 - 0.5 * x * y * y)
    return jnp.where(x > 0.0, y, 0.0)


def _zchunks(rows):
    out, off = [], 0
    while off < rows:
        sz = min(128, rows - off)
        out.append((off, sz))
        off += sz
    return out


def _build_cheb_kernel(np_, f, nec, pool_cfg, iota_shape, split):
    ncores = 2 if split else 1
    fh = f // ncores
    rpt = np_ // _NT
    nr = np_ // 16
    drows = nr // _NT
    zlist = _zchunks(rpt)
    nzc, czs = iota_shape
    nv = fh // 16
    nb = {80: 8, 20: 4, 6: 3}[nec]
    nq = nec // nb

    mesh = plsc.VectorSubcoreMesh(core_axis_name="c", subcore_axis_name="s",
                                  num_cores=ncores)

    scratch = [
        pltpu.VMEM((nec, 128), _i32),
        pltpu.VMEM((nec, 128), _i32),
        pltpu.VMEM((nzc, czs), _i32),
    ] + [pltpu.VMEM((128, fh), _f32) for _ in range(nb)] + [
        pltpu.VMEM((128, fh), _f32),
        pltpu.VMEM((rpt, fh), _f32),
        pltpu.VMEM((rpt, fh), _f32),
        pltpu.VMEM((rpt, fh), _f32),
        pltpu.VMEM((rpt, 16), _f32),
        pltpu.VMEM((nr, 16), _f32),
        pltpu.VMEM_SHARED((np_, fh), _f32),
        pltpu.VMEM_SHARED((nr, 16), _f32),
    ] + [pltpu.SemaphoreType.DMA for _ in range(2 * nb)]
    if pool_cfg is not None:
        np_prev, n_pc, cp = pool_cfg
        rptp = np_prev // _NT
        scratch.append(pltpu.VMEM((n_pc, cp), _i32))

    if split:
        out_type = (jax.ShapeDtypeStruct((2, _KCH, np_, fh), _f32),
                    jax.ShapeDtypeStruct((2, np_, fh), _f32))
    else:
        out_type = (jax.ShapeDtypeStruct((_KCH, np_, fh), _f32),
                    jax.ShapeDtypeStruct((np_, fh), _f32))

    def body(*refs):
        if pool_cfg is None:
            hin, srcp, dstp, iotah, txs, sbuf = refs[:6]
            rest = refs[6:]
            prow_v = None
        else:
            hin, poolrow, srcp, dstp, iotah, txs, sbuf = refs[:7]
            rest = refs[7:]
        src_v, dst_v, iota_v = rest[:3]
        rbs = rest[3:3 + nb]
        zbuf, txA, txB, abuf, dinvb, degp, acc, degsh = rest[3 + nb:11 + nb]
        gsems = rest[11 + nb:11 + 2 * nb]
        ssems = rest[11 + 2 * nb:11 + 3 * nb]
        if pool_cfg is not None:
            prow_v = rest[11 + 3 * nb]
        rb0 = rbs[0]

        wid = lax.axis_index("s")
        if split:
            cid = lax.axis_index("c")
            stab = sbuf.at[cid]

            def tx_dst(k):
                return txs.at[cid, k, pl.ds(wid * rpt, rpt)]

            s_dst = sbuf.at[cid, pl.ds(wid * rpt, rpt)]
        else:
            stab = sbuf

            def tx_dst(k):
                return txs.at[k, pl.ds(wid * rpt, rpt)]

            s_dst = sbuf.at[pl.ds(wid * rpt, rpt)]
        z16f = jnp.zeros((16,), _f32)
        ones16 = jnp.full((16,), 1.0, _f32)

        pltpu.sync_copy(srcp.at[wid], src_v)
        pltpu.sync_copy(dstp.at[wid], dst_v)
        pltpu.sync_copy(iotah, iota_v)
        if pool_cfg is not None:
            pltpu.sync_copy(poolrow.at[wid], prow_v)

        @pl.loop(0, 128, unroll=8)
        def _(r):
            for v in range(nv):
                zbuf[r, pl.ds(16 * v, 16)] = z16f

        @pl.loop(0, nr, unroll=8)
        def _(r):
            degp[r] = z16f

        @pl.loop(0, nec)
        def _(j):
            for g in range(8):
                d = dst_v[j, pl.ds(16 * g, 16)]
                row = jnp.right_shift(d, 4)
                lane = jnp.bitwise_and(d, 15)
                plsc.addupdate_scatter(degp, [row, lane], ones16)

        pltpu.sync_copy(zbuf.at[pl.ds(0, drows), pl.ds(0, 16)],
                        degsh.at[pl.ds(wid * drows, drows)])
        plsc.subcore_barrier()
        for z in range(nzc):
            pltpu.sync_copy(degp.at[pl.ds(z * czs, czs)],
                            degsh.at[iota_v.at[z]], add=True)
        plsc.subcore_barrier()

        pltpu.sync_copy(degsh.at[pl.ds(wid * drows, drows)],
                        degp.at[pl.ds(0, drows)])

        @pl.loop(0, drows, unroll=4)
        def _(r):
            degp[r] = _rsqrt16(degp[r])

        @pl.loop(0, rpt, unroll=8)
        def _(r):
            ir = jnp.full((16,), jnp.right_shift(r, 4), _i32)
            il = jnp.full((16,), jnp.bitwise_and(r, 15), _i32)
            dinvb[r] = plsc.load_gather(degp, [ir, il])

        if pool_cfg is None:
            pltpu.sync_copy(hin.at[pl.ds(wid * rpt, rpt)], txA)
        else:
            for off, sz in zlist:
                pltpu.sync_copy(zbuf.at[pl.ds(0, sz)],
                                acc.at[pl.ds(wid * rpt + off, sz)])
            plsc.subcore_barrier()
            for pc in range(n_pc):
                hsrc = (hin.at[cid, pl.ds(wid * rptp + pc * cp, cp)] if split
                        else hin.at[pl.ds(wid * rptp + pc * cp, cp)])
                pltpu.sync_copy(hsrc, rb0.at[pl.ds(0, cp)])
                pltpu.sync_copy(rb0.at[pl.ds(0, cp)],
                                acc.at[prow_v.at[pc]], add=True)
            plsc.subcore_barrier()
            pltpu.sync_copy(acc.at[pl.ds(wid * rpt, rpt)], txA)

        pltpu.sync_copy(txA, tx_dst(0))

        @pl.loop(0, rpt, unroll=8)
        def _(r):
            d = dinvb[r]
            for v in range(nv):
                sl = pl.ds(16 * v, 16)
                abuf[r, sl] = d * txA[r, sl]

        pltpu.sync_copy(abuf, s_dst)
        plsc.subcore_barrier()

        bufs = [txA, txB]
        for k in range(1, _KCH):
            for off, sz in zlist:
                pltpu.sync_copy(zbuf.at[pl.ds(0, sz)],
                                acc.at[pl.ds(wid * rpt + off, sz)])
            plsc.subcore_barrier()

            for b in range(nb):
                pltpu.async_copy(stab.at[src_v.at[b]], rbs[b], gsems[b])

            @pl.loop(0, nq)
            def _(q):
                base = q * nb
                for b in range(nb):
                    j = base + b
                    pltpu.make_async_copy(stab.at[src_v.at[j]], rbs[b],
                                          gsems[b]).wait()
                    pltpu.sync_copy(rbs[b], acc.at[dst_v.at[j]], add=True)

                    @pl.when(q + 1 < nq)
                    def _(b=b, j=j):
                        pltpu.async_copy(stab.at[src_v.at[j + nb]], rbs[b],
                                         gsems[b])

            plsc.subcore_barrier()

            pltpu.sync_copy(acc.at[pl.ds(wid * rpt, rpt)], abuf)
            tnew = bufs[k % 2]
            last = k == _KCH - 1

            @pl.loop(0, rpt, unroll=8)
            def _(r):
                d = dinvb[r]
                for v in range(nv):
                    sl = pl.ds(16 * v, 16)
                    a = abuf[r, sl]
                    if k == 1:
                        tn = -(d * a)
                    else:
                        tn = (-2.0) * d * a - tnew[r, sl]
                    tnew[r, sl] = tn
                    if not last:
                        abuf[r, sl] = d * tn

            pltpu.sync_copy(tnew, tx_dst(k))
            if not last:
                pltpu.sync_copy(abuf, s_dst)
                plsc.subcore_barrier()

    params = pltpu.CompilerParams(needs_layout_passes=False,
                                  use_tc_tiling_on_sc=False)
    return functools.partial(pl.kernel, out_type=out_type, mesh=mesh,
                             scratch_types=scratch,
                             compiler_params=params)(body)


def _build_dense_kernel(np_, f, fo, bn, split_in):
    grid = (2, np_ // bn)
    foh = fo // 2

    def body(txs_ref, w_ref, b_ref, s_ref, out_ref):
        acc = jnp.zeros((bn, foh), _f32)
        for k in range(_KCH):
            if split_in:
                acc += jnp.dot(txs_ref[0, k], w_ref[0, 0, k],
                               preferred_element_type=_f32)
                acc += jnp.dot(txs_ref[1, k], w_ref[0, 1, k],
                               preferred_element_type=_f32)
            else:
                acc += jnp.dot(txs_ref[k], w_ref[0, k],
                               preferred_element_type=_f32)
        acc = jnp.maximum(acc + b_ref[0], 0.0)
        out_ref[0] = acc * s_ref[...]

    if split_in:
        fin = f // 2
        tx_spec = pl.BlockSpec((2, _KCH, bn, fin), lambda j, i: (0, 0, i, 0))
        w_spec = pl.BlockSpec((1, 2, _KCH, fin, foh),
                              lambda j, i: (j, 0, 0, 0, 0))
    else:
        tx_spec = pl.BlockSpec((_KCH, bn, f), lambda j, i: (0, i, 0))
        w_spec = pl.BlockSpec((1, _KCH, f, foh), lambda j, i: (j, 0, 0, 0))

    return pl.pallas_call(
        body,
        grid=grid,
        in_specs=[
            tx_spec,
            w_spec,
            pl.BlockSpec((1, 1, foh), lambda j, i: (j, 0, 0)),
            pl.BlockSpec((bn, 1), lambda j, i: (i, 0)),
        ],
        out_specs=pl.BlockSpec((1, bn, foh), lambda j, i: (j, i, 0)),
        out_shape=jax.ShapeDtypeStruct((2, np_, foh), _f32),
    )


def _build_final_kernel(np_, f, fo, bn, ncls):
    grid = np_ // bn
    fin = f // 2

    def body(txs_ref, w_ref, b_ref, wl_ref, oh_ref, bl_ref, out_ref):
        i = pl.program_id(0)
        h = jnp.zeros((bn, fo), _f32)
        for k in range(_KCH):
            h += jnp.dot(txs_ref[0, k], w_ref[0, k],
                         preferred_element_type=_f32)
            h += jnp.dot(txs_ref[1, k], w_ref[1, k],
                         preferred_element_type=_f32)
        h = h + b_ref[...]
        z = jnp.zeros((1, 128), _f32)
        for o in range(ncls):
            s = jnp.sum(h * wl_ref[o])
            z += s * oh_ref[o:o + 1, :]

        @pl.when(i == 0)
        def _():
            out_ref[...] = jnp.broadcast_to(bl_ref[...], (8, 128))

        out_ref[...] += jnp.broadcast_to(z, (8, 128))

    return pl.pallas_call(
        body,
        grid=(grid,),
        in_specs=[
            pl.BlockSpec((2, _KCH, bn, fin), lambda i: (0, 0, i, 0)),
            pl.BlockSpec((2, _KCH, fin, fo), lambda i: (0, 0, 0, 0)),
            pl.BlockSpec((1, fo), lambda i: (0, 0)),
            pl.BlockSpec((ncls, bn, fo), lambda i: (0, i, 0)),
            pl.BlockSpec((ncls, 128), lambda i: (0, 0)),
            pl.BlockSpec((1, 128), lambda i: (0, 0)),
        ],
        out_specs=pl.BlockSpec((8, 128), lambda i: (0, 0)),
        out_shape=jax.ShapeDtypeStruct((8, 128), _f32),
    )


def _pad_edges(ei, nec_total, trash, np_):
    e = ei.shape[1]
    ep = _NT * nec_total * 128
    spread = trash + jnp.arange(ep - e, dtype=_i32) % (np_ - trash)
    pad = jnp.stack([spread, spread])
    full = jnp.concatenate([ei, pad], axis=1)
    return (full[0].reshape(_NT, nec_total, 128),
            full[1].reshape(_NT, nec_total, 128))


def _pad_rows(idx, np_prev, n_pc, cp, trash, np_tgt):
    npad = np_prev - idx.shape[0]
    p = trash + jnp.arange(npad, dtype=_i32) % (np_tgt - trash)
    return jnp.concatenate([idx, p]).reshape(_NT, n_pc, cp)


def _pad_val(v, np_prev):
    return jnp.concatenate([v, jnp.zeros((np_prev - v.shape[0],), _f32)])


_cheb0 = _build_cheb_kernel(_NP0, _F0, 80, None, (5, 128), False)
_cheb1 = _build_cheb_kernel(_NP1, _F1, 20, (_NP0, 5, 128), (5, 32), True)
_cheb2 = _build_cheb_kernel(_NP2, _F2, 6, (_NP1, 5, 32), (2, 32), True)
_dense0 = _build_dense_kernel(_NP0, _F0, 64, 1024, False)
_dense1 = _build_dense_kernel(_NP1, _F1, 128, 512, True)
_final = _build_final_kernel(_NP2, _F2, 256, 128, 10)


def kernel(x, edge_index0, edge_index1, edge_index2, D0_row, D0_col, D0_val,
           D1_row, D1_col, D1_val, W0, b0, W1, b1, W2, b2, Wlin, blin):
    n0, n1, n2 = 10000, 2500, 625
    iota0 = jnp.arange(_NP0 // 16, dtype=_i32).reshape(5, 128)
    iota1 = jnp.arange(_NP1 // 16, dtype=_i32).reshape(5, 32)
    iota2 = jnp.arange(_NP2 // 16, dtype=_i32).reshape(2, 32)

    xp = jnp.zeros((_NP0, _F0), _f32).at[:n0, :3].set(x)
    src0, dst0 = _pad_edges(edge_index0, 80, n0, _NP0)
    src1, dst1 = _pad_edges(edge_index1, 20, n1, _NP1)
    src2, dst2 = _pad_edges(edge_index2, 6, n2, _NP2)

    txs0, _ = _cheb0(xp, src0, dst0, iota0)

    w0p = jnp.zeros((_KCH, _F0, 64), _f32).at[:, :3, :].set(W0)
    w0s = jnp.stack([w0p[:, :, :32], w0p[:, :, 32:]])
    b0s = jnp.stack([b0[:32].reshape(1, 32), b0[32:].reshape(1, 32)])
    h0 = _dense0(txs0, w0s, b0s, _pad_val(D0_val, _NP0).reshape(_NP0, 1))

    pr0 = _pad_rows(D0_row, _NP0, 5, 128, n1, _NP1)
    txs1, _ = _cheb1(h0, pr0, src1, dst1, iota1)

    w1q = jnp.stack([
        jnp.stack([W1[:, :32, :64], W1[:, 32:, :64]]),
        jnp.stack([W1[:, :32, 64:], W1[:, 32:, 64:]]),
    ])
    b1s = jnp.stack([b1[:64].reshape(1, 64), b1[64:].reshape(1, 64)])
    h1 = _dense1(txs1, w1q, b1s, _pad_val(D1_val, _NP1).reshape(_NP1, 1))

    pr1 = _pad_rows(D1_row, _NP1, 5, 32, n2, _NP2)
    txs2, _ = _cheb2(h1, pr1, src2, dst2, iota2)

    w2s = jnp.stack([W2[:, :64, :], W2[:, 64:, :]])
    wlt = jnp.transpose(Wlin.reshape(n2, 256, 10), (2, 0, 1))
    wlt = jnp.zeros((10, _NP2, 256), _f32).at[:, :n2, :].set(wlt)
    oh = jnp.eye(10, 128, dtype=_f32)
    bl = jnp.zeros((1, 128), _f32).at[0, :10].set(blin)

    z8 = _final(txs2, w2s, b2.reshape(1, 256), wlt, oh, bl)
    return z8[0, :10]
